# baseline (device time: 12843 ns/iter reference)
import jax
import jax.numpy as jnp
from jax import lax
from jax.experimental import pallas as pl
from jax.experimental.pallas import tpu as pltpu

N_DEV = 4


def kernel(x):
    m_per, n = x.shape

    def body(x_ref, out_ref, send_sems, recv_sems):
        my_x = lax.axis_index("x")
        my_y = lax.axis_index("y")
        my_z = lax.axis_index("z")

        barrier_sem = pltpu.get_barrier_semaphore()

        @pl.when(my_z < N_DEV - 1)
        def _():
            pl.semaphore_signal(
                barrier_sem, inc=1,
                device_id=(my_x, my_y, my_z + 1),
                device_id_type=pl.DeviceIdType.MESH,
            )

        @pl.when(my_z > 0)
        def _():
            pl.semaphore_signal(
                barrier_sem, inc=1,
                device_id=(my_x, my_y, my_z - 1),
                device_id_type=pl.DeviceIdType.MESH,
            )

        my_slice = pl.ds(my_z * m_per, m_per)
        out_ref[my_slice, :] = x_ref[:, :].astype(jnp.bfloat16)

        is_middle = jnp.logical_and(my_z > 0, my_z < N_DEV - 1)

        @pl.when(is_middle)
        def _():
            pl.semaphore_wait(barrier_sem, 2)

        @pl.when(jnp.logical_not(is_middle))
        def _():
            pl.semaphore_wait(barrier_sem, 1)

        def send_chain(z):
            targets = sorted(
                (t for t in range(N_DEV) if t != z),
                key=lambda t: -abs(t - z),
            )
            for i, t in enumerate(targets):
                rdma = pltpu.make_async_remote_copy(
                    src_ref=out_ref.at[my_slice],
                    dst_ref=out_ref.at[my_slice],
                    send_sem=send_sems.at[(t - z) % N_DEV - 1],
                    recv_sem=recv_sems.at[(z - t) % N_DEV - 1],
                    device_id=(my_x, my_y, t),
                    device_id_type=pl.DeviceIdType.MESH,
                )
                rdma.start()
                if i < len(targets) - 1:
                    rdma.wait_send()
                else:
                    last = rdma
            return last

        for z in range(N_DEV):

            @pl.when(my_z == z)
            def _(z=z):
                send_chain(z).wait_send()

        for d in range(1, N_DEV):
            origin = (my_z + d) % N_DEV
            recv = pltpu.make_async_remote_copy(
                src_ref=out_ref.at[my_slice],
                dst_ref=out_ref.at[pl.ds(origin * m_per, m_per)],
                send_sem=send_sems.at[d - 1],
                recv_sem=recv_sems.at[d - 1],
                device_id=(my_x, my_y, origin),
                device_id_type=pl.DeviceIdType.MESH,
            )
            recv.wait_recv()

    return pl.pallas_call(
        body,
        out_shape=jax.ShapeDtypeStruct((N_DEV * m_per, n), jnp.bfloat16),
        in_specs=[pl.BlockSpec(memory_space=pltpu.VMEM)],
        out_specs=pl.BlockSpec(memory_space=pltpu.VMEM),
        scratch_shapes=[
            pltpu.SemaphoreType.DMA((N_DEV - 1,)),
            pltpu.SemaphoreType.DMA((N_DEV - 1,)),
        ],
        compiler_params=pltpu.CompilerParams(collective_id=0),
    )(x)
